# baseline (device time: 2469518 ns/iter reference)
import jax
import jax.numpy as jnp
from jax import lax
from jax.experimental import pallas as pl
from jax.experimental.pallas import tpu as pltpu


def _exchange(logits):
    t, v = logits.shape

    def body(logits_ref, out_ref, copy_sem, send_sem, recv_sem):
        my_x = lax.axis_index("x")
        my_y = lax.axis_index("y")
        my_z = lax.axis_index("z")
        nbr = (my_x, 1 - my_y, my_z)

        barrier_sem = pltpu.get_barrier_semaphore()
        pl.semaphore_signal(
            barrier_sem, inc=1, device_id=nbr,
            device_id_type=pl.DeviceIdType.MESH,
        )
        pl.semaphore_wait(barrier_sem, 1)

        local_cp = pltpu.make_async_copy(
            logits_ref, out_ref.at[my_y], copy_sem
        )
        local_cp.start()

        rdma = pltpu.make_async_remote_copy(
            src_ref=logits_ref,
            dst_ref=out_ref.at[my_y],
            send_sem=send_sem,
            recv_sem=recv_sem,
            device_id=nbr,
            device_id_type=pl.DeviceIdType.MESH,
        )
        rdma.start()
        local_cp.wait()
        rdma.wait()

    return pl.pallas_call(
        body,
        out_shape=jax.ShapeDtypeStruct((2, t, v), logits.dtype),
        in_specs=[pl.BlockSpec(memory_space=pl.ANY)],
        out_specs=pl.BlockSpec(memory_space=pl.ANY),
        scratch_shapes=[
            pltpu.SemaphoreType.DMA,
            pltpu.SemaphoreType.DMA,
            pltpu.SemaphoreType.DMA,
        ],
        compiler_params=pltpu.CompilerParams(collective_id=0),
    )(logits)


def kernel(x, W):
    t, _ = x.shape
    logits = jnp.dot(x, W, preferred_element_type=jnp.float32)
    pair = _exchange(logits)
    m = jnp.max(pair, axis=(0, 2), keepdims=True)
    e = jnp.exp(pair - m)
    d = jnp.sum(e, axis=(0, 2), keepdims=True)
    p = e / d
    return jnp.concatenate([p[0], p[1]], axis=-1).astype(jnp.float32)


# device time: 2469426 ns/iter; 1.0000x vs baseline; 1.0000x over previous
import jax
import jax.numpy as jnp
from jax import lax
from jax.experimental import pallas as pl
from jax.experimental.pallas import tpu as pltpu


def _exchange(logits):
    t, v = logits.shape

    def body(logits_ref, out_ref, copy_sem, send_sem, recv_sem):
        my_x = lax.axis_index("x")
        my_y = lax.axis_index("y")
        my_z = lax.axis_index("z")
        nbr = (my_x, 1 - my_y, my_z)

        barrier_sem = pltpu.get_barrier_semaphore()
        pl.semaphore_signal(
            barrier_sem, inc=1, device_id=nbr,
            device_id_type=pl.DeviceIdType.MESH,
        )
        pl.semaphore_wait(barrier_sem, 1)

        n_ch = 32
        rows = t // n_ch
        copies = []
        rdmas = []
        for i in range(n_ch):
            rs = pl.ds(i * rows, rows)
            cp = pltpu.make_async_copy(
                logits_ref.at[rs], out_ref.at[my_y, rs], copy_sem
            )
            cp.start()
            copies.append(cp)
            rdma = pltpu.make_async_remote_copy(
                src_ref=logits_ref.at[rs],
                dst_ref=out_ref.at[my_y, rs],
                send_sem=send_sem,
                recv_sem=recv_sem,
                device_id=nbr,
                device_id_type=pl.DeviceIdType.MESH,
            )
            rdma.start()
            rdmas.append(rdma)
        for cp in copies:
            cp.wait()
        for rdma in rdmas:
            rdma.wait()

    return pl.pallas_call(
        body,
        out_shape=jax.ShapeDtypeStruct((2, t, v), logits.dtype),
        in_specs=[pl.BlockSpec(memory_space=pl.ANY)],
        out_specs=pl.BlockSpec(memory_space=pl.ANY),
        scratch_shapes=[
            pltpu.SemaphoreType.DMA,
            pltpu.SemaphoreType.DMA,
            pltpu.SemaphoreType.DMA,
        ],
        compiler_params=pltpu.CompilerParams(collective_id=0),
    )(logits)


def kernel(x, W):
    t, _ = x.shape
    logits = jnp.dot(x, W, preferred_element_type=jnp.float32)
    pair = _exchange(logits)
    m = jnp.max(pair, axis=(0, 2), keepdims=True)
    e = jnp.exp(pair - m)
    d = jnp.sum(e, axis=(0, 2), keepdims=True)
    p = e / d
    return jnp.concatenate([p[0], p[1]], axis=-1).astype(jnp.float32)


# device time: 1133330 ns/iter; 2.1790x vs baseline; 2.1789x over previous
import jax
import jax.numpy as jnp
from jax import lax
from jax.experimental import pallas as pl
from jax.experimental.pallas import tpu as pltpu


def _exchange(logits):
    t, v = logits.shape

    def body(logits_ref, out_ref, send_sem, recv_sem):
        my_x = lax.axis_index("x")
        my_y = lax.axis_index("y")
        my_z = lax.axis_index("z")
        nbr = (my_x, 1 - my_y, my_z)

        barrier_sem = pltpu.get_barrier_semaphore()
        pl.semaphore_signal(
            barrier_sem, inc=1, device_id=nbr,
            device_id_type=pl.DeviceIdType.MESH,
        )
        pl.semaphore_wait(barrier_sem, 1)

        rdma = pltpu.make_async_remote_copy(
            src_ref=logits_ref,
            dst_ref=out_ref,
            send_sem=send_sem,
            recv_sem=recv_sem,
            device_id=nbr,
            device_id_type=pl.DeviceIdType.MESH,
        )
        rdma.start()
        rdma.wait()

    return pl.pallas_call(
        body,
        out_shape=jax.ShapeDtypeStruct((t, v), logits.dtype),
        in_specs=[pl.BlockSpec(memory_space=pl.ANY)],
        out_specs=pl.BlockSpec(memory_space=pl.ANY),
        scratch_shapes=[
            pltpu.SemaphoreType.DMA,
            pltpu.SemaphoreType.DMA,
        ],
        compiler_params=pltpu.CompilerParams(collective_id=0),
    )(logits)


def kernel(x, W):
    my_y = lax.axis_index("y")
    logits = jnp.dot(x, W, preferred_element_type=jnp.float32)
    remote = _exchange(logits)

    m = jnp.maximum(
        jnp.max(logits, axis=-1, keepdims=True),
        jnp.max(remote, axis=-1, keepdims=True),
    )
    e_loc = jnp.exp(logits - m)
    e_rem = jnp.exp(remote - m)
    d = jnp.sum(e_loc, axis=-1, keepdims=True) + jnp.sum(
        e_rem, axis=-1, keepdims=True
    )
    p_loc = e_loc / d
    p_rem = e_rem / d
    return lax.cond(
        my_y == 0,
        lambda a, b: jnp.concatenate([a, b], axis=-1),
        lambda a, b: jnp.concatenate([b, a], axis=-1),
        p_loc,
        p_rem,
    ).astype(jnp.float32)


# device time: 902462 ns/iter; 2.7364x vs baseline; 1.2558x over previous
import jax
import jax.numpy as jnp
from jax import lax
from jax.experimental import pallas as pl
from jax.experimental.pallas import tpu as pltpu

_N_CH = 16


def _fused_exchange_softmax(logits):
    t, v = logits.shape
    rows = t // _N_CH

    def body(
        logits_ref,
        final_ref,
        comm_ref,
        lbuf, rbuf,
        send_sem,
        recv_sems,
        lsem, rsem, osem,
    ):
        my_x = lax.axis_index("x")
        my_y = lax.axis_index("y")
        my_z = lax.axis_index("z")
        nbr = (my_x, 1 - my_y, my_z)

        barrier_sem = pltpu.get_barrier_semaphore()
        pl.semaphore_signal(
            barrier_sem, inc=1, device_id=nbr,
            device_id_type=pl.DeviceIdType.MESH,
        )
        pl.semaphore_wait(barrier_sem, 1)

        rdmas = []
        for i in range(_N_CH):
            rs = pl.ds(i * rows, rows)
            r = pltpu.make_async_remote_copy(
                src_ref=logits_ref.at[rs],
                dst_ref=comm_ref.at[rs],
                send_sem=send_sem,
                recv_sem=recv_sems.at[i],
                device_id=nbr,
                device_id_type=pl.DeviceIdType.MESH,
            )
            r.start()
            rdmas.append(r)

        for i in range(_N_CH):
            rs = pl.ds(i * rows, rows)
            cp_l = pltpu.make_async_copy(logits_ref.at[rs], lbuf, lsem)
            cp_l.start()
            rdmas[i].wait_recv()
            cp_r = pltpu.make_async_copy(comm_ref.at[rs], rbuf, rsem)
            cp_r.start()
            cp_l.wait()
            cp_r.wait()

            l = lbuf[...]
            r = rbuf[...]
            m = jnp.maximum(
                jnp.max(l, axis=1, keepdims=True),
                jnp.max(r, axis=1, keepdims=True),
            )
            e_l = jnp.exp(l - m)
            e_r = jnp.exp(r - m)
            inv = 1.0 / (
                jnp.sum(e_l, axis=1, keepdims=True)
                + jnp.sum(e_r, axis=1, keepdims=True)
            )
            lbuf[...] = e_l * inv
            rbuf[...] = e_r * inv

            st_l = pltpu.make_async_copy(
                lbuf, final_ref.at[rs, pl.ds(my_y * v, v)], osem
            )
            st_l.start()
            st_r = pltpu.make_async_copy(
                rbuf, final_ref.at[rs, pl.ds((1 - my_y) * v, v)], osem
            )
            st_r.start()
            st_l.wait()
            st_r.wait()

        for r in rdmas:
            r.wait_send()

    final, _ = pl.pallas_call(
        body,
        out_shape=(
            jax.ShapeDtypeStruct((t, 2 * v), logits.dtype),
            jax.ShapeDtypeStruct((t, v), logits.dtype),
        ),
        in_specs=[pl.BlockSpec(memory_space=pl.ANY)],
        out_specs=(
            pl.BlockSpec(memory_space=pl.ANY),
            pl.BlockSpec(memory_space=pl.ANY),
        ),
        scratch_shapes=[
            pltpu.VMEM((rows, v), jnp.float32),
            pltpu.VMEM((rows, v), jnp.float32),
            pltpu.SemaphoreType.DMA,
            pltpu.SemaphoreType.DMA((_N_CH,)),
            pltpu.SemaphoreType.DMA,
            pltpu.SemaphoreType.DMA,
            pltpu.SemaphoreType.DMA,
        ],
        compiler_params=pltpu.CompilerParams(collective_id=0),
    )(logits)
    return final


def kernel(x, W):
    logits = jnp.dot(x, W, preferred_element_type=jnp.float32)
    return _fused_exchange_softmax(logits)


# device time: 558739 ns/iter; 4.4198x vs baseline; 1.6152x over previous
import jax
import jax.numpy as jnp
from jax import lax
from jax.experimental import pallas as pl
from jax.experimental.pallas import tpu as pltpu

_N_CH = 16


def _fused_exchange_softmax(logits, logits_bf16):
    t, v = logits.shape
    rows = t // _N_CH

    def body(
        logits_ref,
        lb_ref,
        final_ref,
        comm_ref,
        lbuf,
        rbuf,
        obuf,
        send_sem,
        recv_sems,
        lsem, rsem, osem,
    ):
        my_x = lax.axis_index("x")
        my_y = lax.axis_index("y")
        my_z = lax.axis_index("z")
        nbr = (my_x, 1 - my_y, my_z)

        barrier_sem = pltpu.get_barrier_semaphore()
        pl.semaphore_signal(
            barrier_sem, inc=1, device_id=nbr,
            device_id_type=pl.DeviceIdType.MESH,
        )
        pl.semaphore_wait(barrier_sem, 1)

        rdmas = []
        for i in range(_N_CH):
            rs = pl.ds(i * rows, rows)
            r = pltpu.make_async_remote_copy(
                src_ref=lb_ref.at[rs],
                dst_ref=comm_ref.at[rs],
                send_sem=send_sem,
                recv_sem=recv_sems.at[i],
                device_id=nbr,
                device_id_type=pl.DeviceIdType.MESH,
            )
            r.start()
            rdmas.append(r)

        for i in range(_N_CH):
            rs = pl.ds(i * rows, rows)
            cp_l = pltpu.make_async_copy(logits_ref.at[rs], lbuf, lsem)
            cp_l.start()
            rdmas[i].wait_recv()
            cp_r = pltpu.make_async_copy(comm_ref.at[rs], rbuf, rsem)
            cp_r.start()
            cp_l.wait()
            cp_r.wait()

            l = lbuf[...]
            r = rbuf[...].astype(jnp.float32)
            m = jnp.maximum(
                jnp.max(l, axis=1, keepdims=True),
                jnp.max(r, axis=1, keepdims=True),
            )
            e_l = jnp.exp(l - m)
            e_r = jnp.exp(r - m)
            inv = 1.0 / (
                jnp.sum(e_l, axis=1, keepdims=True)
                + jnp.sum(e_r, axis=1, keepdims=True)
            )
            lbuf[...] = e_l * inv
            obuf[...] = e_r * inv

            st_l = pltpu.make_async_copy(
                lbuf, final_ref.at[rs, pl.ds(my_y * v, v)], osem
            )
            st_l.start()
            st_r = pltpu.make_async_copy(
                obuf, final_ref.at[rs, pl.ds((1 - my_y) * v, v)], osem
            )
            st_r.start()
            st_l.wait()
            st_r.wait()

        for r in rdmas:
            r.wait_send()

    final, _ = pl.pallas_call(
        body,
        out_shape=(
            jax.ShapeDtypeStruct((t, 2 * v), logits.dtype),
            jax.ShapeDtypeStruct((t, v), jnp.bfloat16),
        ),
        in_specs=[
            pl.BlockSpec(memory_space=pl.ANY),
            pl.BlockSpec(memory_space=pl.ANY),
        ],
        out_specs=(
            pl.BlockSpec(memory_space=pl.ANY),
            pl.BlockSpec(memory_space=pl.ANY),
        ),
        scratch_shapes=[
            pltpu.VMEM((rows, v), jnp.float32),
            pltpu.VMEM((rows, v), jnp.bfloat16),
            pltpu.VMEM((rows, v), jnp.float32),
            pltpu.SemaphoreType.DMA,
            pltpu.SemaphoreType.DMA((_N_CH,)),
            pltpu.SemaphoreType.DMA,
            pltpu.SemaphoreType.DMA,
            pltpu.SemaphoreType.DMA,
        ],
        compiler_params=pltpu.CompilerParams(collective_id=0),
    )(logits, logits_bf16)
    return final


def kernel(x, W):
    logits = jnp.dot(x, W, preferred_element_type=jnp.float32)
    return _fused_exchange_softmax(logits, logits.astype(jnp.bfloat16))
